# baseline (speedup 1.0000x reference)
import functools

import jax
import jax.numpy as jnp
from jax import lax
from jax.experimental import pallas as pl
from jax.experimental.pallas import tpu as pltpu
from jax.experimental.pallas import tpu_sc as plsc

ANOMALY_THRESHOLD = 0.5
CONTRAST_EPS = 1e-08

NUM_CORES = 2
NUM_SUBCORES = 16
NUM_WORKERS = NUM_CORES * NUM_SUBCORES
LANES = 16

SLAB = 96 * 96
ROW_SLABS = 96
TC_SLABS = 48
TC_CHUNK = 8
CH_SLABS = 1
NBUF = 4
ROW_VREGS = 96 // LANES


def _sc_partials_body(sc_per_worker, pred_hbm, tgt_hbm, out_hbm,
                      pbuf, tbuf, stage, sem_p0, sem_t0, sem_p1, sem_t1,
                      sem_p2, sem_t2, sem_p3, sem_t3):
    cid = lax.axis_index("c")
    sid = lax.axis_index("s")
    wid = sid * NUM_CORES + cid
    row = wid // 2
    hlf = wid % 2
    base = row * ROW_SLABS + TC_SLABS + hlf * sc_per_worker
    nchunk = sc_per_worker // CH_SLABS

    sems = ((sem_p0, sem_t0), (sem_p1, sem_t1), (sem_p2, sem_t2),
            (sem_p3, sem_t3))

    def start(k):
        b = k % NBUF
        sl = pl.ds(base + k * CH_SLABS, CH_SLABS)
        cp_p = pltpu.make_async_copy(pred_hbm.at[sl], pbuf.at[b], sems[b][0])
        cp_t = pltpu.make_async_copy(tgt_hbm.at[sl], tbuf.at[b], sems[b][1])
        cp_p.start()
        cp_t.start()
        return cp_p, cp_t

    zero = jnp.zeros((LANES,), jnp.float32)
    ones = jnp.ones((LANES,), jnp.float32)
    acc = (zero, zero, zero, zero, zero)

    def tree(xs):
        while len(xs) > 1:
            nxt = [xs[i] + xs[i + 1] for i in range(0, len(xs) - 1, 2)]
            if len(xs) % 2:
                nxt.append(xs[-1])
            xs = nxt
        return xs[0]

    pend = [start(k) for k in range(NBUF - 1)]
    for k in range(nchunk):
        b = k % NBUF
        cp = pend.pop(0)
        cp[0].wait()
        cp[1].wait()
        if k + NBUF - 1 < nchunk:
            pend.append(start(k + NBUF - 1))

        def chunk_body(r, carry, b=b):
            c_cnt, c_spa, c_sta, c_sp, c_st = carry
            ms, pms, tms, ps, ts = [], [], [], [], []
            for s in range(CH_SLABS):
                for l in range(ROW_VREGS):
                    p = pbuf[b, s, r, pl.ds(l * LANES, LANES)]
                    t = tbuf[b, s, r, pl.ds(l * LANES, LANES)]
                    anom = t > ANOMALY_THRESHOLD
                    ms.append(jnp.where(anom, ones, zero))
                    pms.append(jnp.where(anom, p, zero))
                    tms.append(jnp.where(anom, t, zero))
                    ps.append(p)
                    ts.append(t)
            return (c_cnt + tree(ms), c_spa + tree(pms), c_sta + tree(tms),
                    c_sp + tree(ps), c_st + tree(ts))

        ch = lax.fori_loop(0, 96, chunk_body, (zero, zero, zero, zero, zero))
        acc = tuple(a + c for a, c in zip(acc, ch))

    for q in range(8):
        for l in range(128 // LANES):
            stage[q, pl.ds(l * LANES, LANES)] = zero
    for q, v in enumerate(acc):
        stage[q, pl.ds(0, LANES)] = v

    pltpu.sync_copy(stage, out_hbm.at[hlf, row])


def _tc_partials_body(nch, p_ref, t_ref, o_ref, acc_ref):
    k = pl.program_id(1)
    p = p_ref[0]
    t = t_ref[0]
    m = (t > ANOMALY_THRESHOLD).astype(jnp.float32)
    parts = (m.sum(axis=0), (p * m).sum(axis=0), (t * m).sum(axis=0),
             p.sum(axis=0), t.sum(axis=0))

    for q, part in enumerate(parts):
        @pl.when(k == 0)
        def _(q=q, part=part):
            acc_ref[q] = part

        @pl.when(k > 0)
        def _(q=q, part=part):
            acc_ref[q] = acc_ref[q] + part

    @pl.when(k == nch - 1)
    def _():
        o_ref[...] = acc_ref[...].sum(axis=1).reshape(1, 5, 96)


def _combine_body(n_per_row, sc_ref, tc_ref, o_ref):
    x = sc_ref[...]
    y_sc = (x[0] + x[1])[:, :5, :LANES].sum(axis=-1)
    y_tc = tc_ref[...].sum(axis=-1)
    y = y_sc + y_tc
    cnt = y[:, 0:1]
    spa = y[:, 1:2]
    sta = y[:, 2:3]
    sp = y[:, 3:4]
    st = y[:, 4:5]
    n = jnp.float32(n_per_row)
    b_cnt = n - cnt
    safe_a = jnp.maximum(cnt, 1.0)
    safe_b = jnp.maximum(b_cnt, 1.0)
    pred_anom_mean = spa / safe_a
    pred_bg_mean = (sp - spa) / safe_b
    tgt_anom_mean = sta / safe_a
    tgt_bg_mean = (st - sta) / safe_b
    valid = (cnt > 0.0) & (b_cnt > 0.0)
    pred_contrast = pred_anom_mean - pred_bg_mean
    tgt_contrast = tgt_anom_mean - tgt_bg_mean
    ratio = pred_contrast / (tgt_contrast + CONTRAST_EPS)
    vf = valid.astype(jnp.float32)
    n_valid = jnp.sum(vf)
    mean_ratio = jnp.sum(ratio * vf) / jnp.maximum(n_valid, 1.0)
    res = jnp.where(n_valid > 0.0, mean_ratio, jnp.float32(1.0))
    o_ref[...] = jnp.full((1, 1), res, jnp.float32)


def kernel(pred, target):
    B, C = pred.shape[0], pred.shape[1]
    total = pred.size
    n_per_row = total // (B * C)
    n_slabs = total // SLAB
    n_rows = B * C
    assert n_slabs == n_rows * ROW_SLABS
    sc_per_worker = (ROW_SLABS - TC_SLABS) // 2
    assert sc_per_worker * 2 == ROW_SLABS - TC_SLABS
    assert sc_per_worker % CH_SLABS == 0
    assert TC_SLABS % TC_CHUNK == 0
    assert n_rows * 2 == NUM_WORKERS

    pf = pred.reshape(n_slabs, 96, 96)
    tf = target.reshape(n_slabs, 96, 96)
    p4 = pred.reshape(n_rows, ROW_SLABS, 96, 96)
    t4 = target.reshape(n_rows, ROW_SLABS, 96, 96)

    mesh = plsc.VectorSubcoreMesh(core_axis_name="c", subcore_axis_name="s")
    sc_fn = pl.kernel(
        functools.partial(_sc_partials_body, sc_per_worker),
        mesh=mesh,
        out_type=jax.ShapeDtypeStruct((2, NUM_SUBCORES, 8, 128),
                                      jnp.float32),
        compiler_params=pltpu.CompilerParams(use_tc_tiling_on_sc=True),
        scratch_types=[
            pltpu.VMEM((NBUF, CH_SLABS, 96, 96), jnp.float32),
            pltpu.VMEM((NBUF, CH_SLABS, 96, 96), jnp.float32),
            pltpu.VMEM((8, 128), jnp.float32),
            pltpu.SemaphoreType.DMA,
            pltpu.SemaphoreType.DMA,
            pltpu.SemaphoreType.DMA,
            pltpu.SemaphoreType.DMA,
            pltpu.SemaphoreType.DMA,
            pltpu.SemaphoreType.DMA,
            pltpu.SemaphoreType.DMA,
            pltpu.SemaphoreType.DMA,
        ],
    )
    sc_partials = sc_fn(pf, tf)

    nch = TC_SLABS // TC_CHUNK
    tc_partials = pl.pallas_call(
        functools.partial(_tc_partials_body, nch),
        grid=(n_rows, nch),
        in_specs=[
            pl.BlockSpec((1, TC_CHUNK, 96, 96), lambda r, k: (r, k, 0, 0)),
            pl.BlockSpec((1, TC_CHUNK, 96, 96), lambda r, k: (r, k, 0, 0)),
        ],
        out_specs=pl.BlockSpec((1, 5, 96), lambda r, k: (r, 0, 0)),
        out_shape=jax.ShapeDtypeStruct((n_rows, 5, 96), jnp.float32),
        scratch_shapes=[pltpu.VMEM((5, 96, 96), jnp.float32)],
    )(p4, t4)

    out = pl.pallas_call(
        functools.partial(_combine_body, n_per_row),
        out_shape=jax.ShapeDtypeStruct((1, 1), jnp.float32),
    )(sc_partials, tc_partials)
    return out[0, 0]


# speedup vs baseline: 1.3894x; 1.3894x over previous
import functools

import jax
import jax.numpy as jnp
from jax import lax
from jax.experimental import pallas as pl
from jax.experimental.pallas import tpu as pltpu
from jax.experimental.pallas import tpu_sc as plsc

ANOMALY_THRESHOLD = 0.5
CONTRAST_EPS = 1e-08

NUM_CORES = 2
NUM_SUBCORES = 16
NUM_WORKERS = NUM_CORES * NUM_SUBCORES
LANES = 16

SLAB = 96 * 96
ROW_SLABS = 96
TC_SLABS = 48
TC_CHUNK = 48
CH_SLABS = 1
NBUF = 4
ROW_VREGS = 96 // LANES


def _sc_partials_body(sc_per_worker, pred_hbm, tgt_hbm, out_hbm,
                      pbuf, tbuf, stage, sem_p0, sem_t0, sem_p1, sem_t1,
                      sem_p2, sem_t2, sem_p3, sem_t3):
    cid = lax.axis_index("c")
    sid = lax.axis_index("s")
    wid = sid * NUM_CORES + cid
    row = wid // 2
    hlf = wid % 2
    base = row * ROW_SLABS + TC_SLABS + hlf * sc_per_worker
    nchunk = sc_per_worker // CH_SLABS

    sems = ((sem_p0, sem_t0), (sem_p1, sem_t1), (sem_p2, sem_t2),
            (sem_p3, sem_t3))

    def start(k):
        b = k % NBUF
        sl = pl.ds(base + k * CH_SLABS, CH_SLABS)
        cp_p = pltpu.make_async_copy(pred_hbm.at[sl], pbuf.at[b], sems[b][0])
        cp_t = pltpu.make_async_copy(tgt_hbm.at[sl], tbuf.at[b], sems[b][1])
        cp_p.start()
        cp_t.start()
        return cp_p, cp_t

    zero = jnp.zeros((LANES,), jnp.float32)
    ones = jnp.ones((LANES,), jnp.float32)
    acc = (zero, zero, zero, zero, zero)

    def tree(xs):
        while len(xs) > 1:
            nxt = [xs[i] + xs[i + 1] for i in range(0, len(xs) - 1, 2)]
            if len(xs) % 2:
                nxt.append(xs[-1])
            xs = nxt
        return xs[0]

    pend = [start(k) for k in range(NBUF - 1)]
    for k in range(nchunk):
        b = k % NBUF
        cp = pend.pop(0)
        cp[0].wait()
        cp[1].wait()
        if k + NBUF - 1 < nchunk:
            pend.append(start(k + NBUF - 1))

        def chunk_body(r, carry, b=b):
            c_cnt, c_spa, c_sta, c_sp, c_st = carry
            ms, pms, tms, ps, ts = [], [], [], [], []
            for s in range(CH_SLABS):
                for l in range(ROW_VREGS):
                    p = pbuf[b, s, r, pl.ds(l * LANES, LANES)]
                    t = tbuf[b, s, r, pl.ds(l * LANES, LANES)]
                    anom = t > ANOMALY_THRESHOLD
                    ms.append(jnp.where(anom, ones, zero))
                    pms.append(jnp.where(anom, p, zero))
                    tms.append(jnp.where(anom, t, zero))
                    ps.append(p)
                    ts.append(t)
            return (c_cnt + tree(ms), c_spa + tree(pms), c_sta + tree(tms),
                    c_sp + tree(ps), c_st + tree(ts))

        ch = lax.fori_loop(0, 96, chunk_body, (zero, zero, zero, zero, zero))
        acc = tuple(a + c for a, c in zip(acc, ch))

    for q in range(8):
        for l in range(128 // LANES):
            stage[q, pl.ds(l * LANES, LANES)] = zero
    for q, v in enumerate(acc):
        stage[q, pl.ds(0, LANES)] = v

    pltpu.sync_copy(stage, out_hbm.at[hlf, row])


def _tc_partials_body(nch, p_ref, t_ref, o_ref, acc_ref):
    k = pl.program_id(1)
    p = p_ref[0]
    t = t_ref[0]
    m = (t > ANOMALY_THRESHOLD).astype(jnp.float32)
    parts = (m.sum(axis=0), (p * m).sum(axis=0), (t * m).sum(axis=0),
             p.sum(axis=0), t.sum(axis=0))

    for q, part in enumerate(parts):
        @pl.when(k == 0)
        def _(q=q, part=part):
            acc_ref[q] = part

        @pl.when(k > 0)
        def _(q=q, part=part):
            acc_ref[q] = acc_ref[q] + part

    @pl.when(k == nch - 1)
    def _():
        o_ref[...] = acc_ref[...].sum(axis=1).reshape(1, 5, 96)


def _combine_body(n_per_row, sc_ref, tc_ref, o_ref):
    x = sc_ref[...]
    y_sc = (x[0] + x[1])[:, :5, :LANES].sum(axis=-1)
    y_tc = tc_ref[...].sum(axis=-1)
    y = y_sc + y_tc
    cnt = y[:, 0:1]
    spa = y[:, 1:2]
    sta = y[:, 2:3]
    sp = y[:, 3:4]
    st = y[:, 4:5]
    n = jnp.float32(n_per_row)
    b_cnt = n - cnt
    safe_a = jnp.maximum(cnt, 1.0)
    safe_b = jnp.maximum(b_cnt, 1.0)
    pred_anom_mean = spa / safe_a
    pred_bg_mean = (sp - spa) / safe_b
    tgt_anom_mean = sta / safe_a
    tgt_bg_mean = (st - sta) / safe_b
    valid = (cnt > 0.0) & (b_cnt > 0.0)
    pred_contrast = pred_anom_mean - pred_bg_mean
    tgt_contrast = tgt_anom_mean - tgt_bg_mean
    ratio = pred_contrast / (tgt_contrast + CONTRAST_EPS)
    vf = valid.astype(jnp.float32)
    n_valid = jnp.sum(vf)
    mean_ratio = jnp.sum(ratio * vf) / jnp.maximum(n_valid, 1.0)
    res = jnp.where(n_valid > 0.0, mean_ratio, jnp.float32(1.0))
    o_ref[...] = jnp.full((1, 1), res, jnp.float32)


def kernel(pred, target):
    B, C = pred.shape[0], pred.shape[1]
    total = pred.size
    n_per_row = total // (B * C)
    n_slabs = total // SLAB
    n_rows = B * C
    assert n_slabs == n_rows * ROW_SLABS
    sc_per_worker = (ROW_SLABS - TC_SLABS) // 2
    assert sc_per_worker * 2 == ROW_SLABS - TC_SLABS
    assert sc_per_worker % CH_SLABS == 0
    assert TC_SLABS % TC_CHUNK == 0
    assert n_rows * 2 == NUM_WORKERS

    pf = pred.reshape(n_slabs, 96, 96)
    tf = target.reshape(n_slabs, 96, 96)
    p4 = pred.reshape(n_rows, ROW_SLABS, 96, 96)
    t4 = target.reshape(n_rows, ROW_SLABS, 96, 96)

    mesh = plsc.VectorSubcoreMesh(core_axis_name="c", subcore_axis_name="s")
    sc_fn = pl.kernel(
        functools.partial(_sc_partials_body, sc_per_worker),
        mesh=mesh,
        out_type=jax.ShapeDtypeStruct((2, NUM_SUBCORES, 8, 128),
                                      jnp.float32),
        compiler_params=pltpu.CompilerParams(use_tc_tiling_on_sc=True),
        scratch_types=[
            pltpu.VMEM((NBUF, CH_SLABS, 96, 96), jnp.float32),
            pltpu.VMEM((NBUF, CH_SLABS, 96, 96), jnp.float32),
            pltpu.VMEM((8, 128), jnp.float32),
            pltpu.SemaphoreType.DMA,
            pltpu.SemaphoreType.DMA,
            pltpu.SemaphoreType.DMA,
            pltpu.SemaphoreType.DMA,
            pltpu.SemaphoreType.DMA,
            pltpu.SemaphoreType.DMA,
            pltpu.SemaphoreType.DMA,
            pltpu.SemaphoreType.DMA,
        ],
    )
    sc_partials = sc_fn(pf, tf)

    nch = TC_SLABS // TC_CHUNK
    tc_partials = pl.pallas_call(
        functools.partial(_tc_partials_body, nch),
        grid=(n_rows, nch),
        in_specs=[
            pl.BlockSpec((1, TC_CHUNK, 96, 96), lambda r, k: (r, k, 0, 0)),
            pl.BlockSpec((1, TC_CHUNK, 96, 96), lambda r, k: (r, k, 0, 0)),
        ],
        out_specs=pl.BlockSpec((1, 5, 96), lambda r, k: (r, 0, 0)),
        out_shape=jax.ShapeDtypeStruct((n_rows, 5, 96), jnp.float32),
        scratch_shapes=[pltpu.VMEM((5, 96, 96), jnp.float32)],
    )(p4, t4)

    out = pl.pallas_call(
        functools.partial(_combine_body, n_per_row),
        out_shape=jax.ShapeDtypeStruct((1, 1), jnp.float32),
    )(sc_partials, tc_partials)
    return out[0, 0]


# speedup vs baseline: 1.4295x; 1.0289x over previous
import functools

import jax
import jax.numpy as jnp
from jax import lax
from jax.experimental import pallas as pl
from jax.experimental.pallas import tpu as pltpu
from jax.experimental.pallas import tpu_sc as plsc

ANOMALY_THRESHOLD = 0.5
CONTRAST_EPS = 1e-08

NUM_CORES = 2
NUM_SUBCORES = 16
NUM_WORKERS = NUM_CORES * NUM_SUBCORES
LANES = 16

SLAB = 96 * 96
ROW_SLABS = 96
TC_SLABS = 56
TC_CHUNK = 56
CH_SLABS = 1
NBUF = 4
ROW_VREGS = 96 // LANES


def _sc_partials_body(sc_per_worker, pred_hbm, tgt_hbm, out_hbm,
                      pbuf, tbuf, stage, sem_p0, sem_t0, sem_p1, sem_t1,
                      sem_p2, sem_t2, sem_p3, sem_t3):
    cid = lax.axis_index("c")
    sid = lax.axis_index("s")
    wid = sid * NUM_CORES + cid
    row = wid // 2
    hlf = wid % 2
    base = row * ROW_SLABS + TC_SLABS + hlf * sc_per_worker
    nchunk = sc_per_worker // CH_SLABS

    sems = ((sem_p0, sem_t0), (sem_p1, sem_t1), (sem_p2, sem_t2),
            (sem_p3, sem_t3))

    def start(k):
        b = k % NBUF
        sl = pl.ds(base + k * CH_SLABS, CH_SLABS)
        cp_p = pltpu.make_async_copy(pred_hbm.at[sl], pbuf.at[b], sems[b][0])
        cp_t = pltpu.make_async_copy(tgt_hbm.at[sl], tbuf.at[b], sems[b][1])
        cp_p.start()
        cp_t.start()
        return cp_p, cp_t

    zero = jnp.zeros((LANES,), jnp.float32)
    ones = jnp.ones((LANES,), jnp.float32)
    acc = (zero, zero, zero, zero, zero)

    def tree(xs):
        while len(xs) > 1:
            nxt = [xs[i] + xs[i + 1] for i in range(0, len(xs) - 1, 2)]
            if len(xs) % 2:
                nxt.append(xs[-1])
            xs = nxt
        return xs[0]

    def chunk_body(b):
        def body(r, carry):
            c_cnt, c_spa, c_sta, c_sp, c_st = carry
            ms, pms, tms, ps, ts = [], [], [], [], []
            for s in range(CH_SLABS):
                for l in range(ROW_VREGS):
                    p = pbuf[b, s, r, pl.ds(l * LANES, LANES)]
                    t = tbuf[b, s, r, pl.ds(l * LANES, LANES)]
                    anom = t > ANOMALY_THRESHOLD
                    ms.append(jnp.where(anom, ones, zero))
                    pms.append(jnp.where(anom, p, zero))
                    tms.append(jnp.where(anom, t, zero))
                    ps.append(p)
                    ts.append(t)
            return (c_cnt + tree(ms), c_spa + tree(pms), c_sta + tree(tms),
                    c_sp + tree(ps), c_st + tree(ts))
        return body

    for k in range(NBUF - 1):
        start(k)

    def group_body(g, carry):
        acc = carry
        for b in range(NBUF):
            k = g * NBUF + b
            cp_p = pltpu.make_async_copy(
                pred_hbm.at[pl.ds(base + k * CH_SLABS, CH_SLABS)],
                pbuf.at[b], sems[b][0])
            cp_t = pltpu.make_async_copy(
                tgt_hbm.at[pl.ds(base + k * CH_SLABS, CH_SLABS)],
                tbuf.at[b], sems[b][1])
            cp_p.wait()
            cp_t.wait()

            k_next = k + NBUF - 1

            @pl.when(k_next < nchunk)
            def _(k_next=k_next, b_next=(b + NBUF - 1) % NBUF):
                sl = pl.ds(base + k_next * CH_SLABS, CH_SLABS)
                pltpu.make_async_copy(pred_hbm.at[sl], pbuf.at[b_next],
                                      sems[b_next][0]).start()
                pltpu.make_async_copy(tgt_hbm.at[sl], tbuf.at[b_next],
                                      sems[b_next][1]).start()

            ch = lax.fori_loop(0, 96, chunk_body(b),
                               (zero, zero, zero, zero, zero))
            acc = tuple(a + c for a, c in zip(acc, ch))
        return acc

    assert nchunk % NBUF == 0
    acc = lax.fori_loop(0, nchunk // NBUF, group_body, acc)

    for q in range(8):
        for l in range(128 // LANES):
            stage[q, pl.ds(l * LANES, LANES)] = zero
    for q, v in enumerate(acc):
        stage[q, pl.ds(0, LANES)] = v

    pltpu.sync_copy(stage, out_hbm.at[hlf, row])


def _tc_partials_body(nch, p_ref, t_ref, o_ref, acc_ref):
    k = pl.program_id(1)
    p = p_ref[0]
    t = t_ref[0]
    m = (t > ANOMALY_THRESHOLD).astype(jnp.float32)
    parts = (m.sum(axis=0), (p * m).sum(axis=0), (t * m).sum(axis=0),
             p.sum(axis=0), t.sum(axis=0))

    for q, part in enumerate(parts):
        @pl.when(k == 0)
        def _(q=q, part=part):
            acc_ref[q] = part

        @pl.when(k > 0)
        def _(q=q, part=part):
            acc_ref[q] = acc_ref[q] + part

    @pl.when(k == nch - 1)
    def _():
        o_ref[...] = acc_ref[...].sum(axis=1).reshape(1, 5, 96)


def _combine_body(n_per_row, sc_ref, tc_ref, o_ref):
    x = sc_ref[...]
    y_sc = (x[0] + x[1])[:, :5, :LANES].sum(axis=-1)
    y_tc = tc_ref[...].sum(axis=-1)
    y = y_sc + y_tc
    cnt = y[:, 0:1]
    spa = y[:, 1:2]
    sta = y[:, 2:3]
    sp = y[:, 3:4]
    st = y[:, 4:5]
    n = jnp.float32(n_per_row)
    b_cnt = n - cnt
    safe_a = jnp.maximum(cnt, 1.0)
    safe_b = jnp.maximum(b_cnt, 1.0)
    pred_anom_mean = spa / safe_a
    pred_bg_mean = (sp - spa) / safe_b
    tgt_anom_mean = sta / safe_a
    tgt_bg_mean = (st - sta) / safe_b
    valid = (cnt > 0.0) & (b_cnt > 0.0)
    pred_contrast = pred_anom_mean - pred_bg_mean
    tgt_contrast = tgt_anom_mean - tgt_bg_mean
    ratio = pred_contrast / (tgt_contrast + CONTRAST_EPS)
    vf = valid.astype(jnp.float32)
    n_valid = jnp.sum(vf)
    mean_ratio = jnp.sum(ratio * vf) / jnp.maximum(n_valid, 1.0)
    res = jnp.where(n_valid > 0.0, mean_ratio, jnp.float32(1.0))
    o_ref[...] = jnp.full((1, 1), res, jnp.float32)


def kernel(pred, target):
    B, C = pred.shape[0], pred.shape[1]
    total = pred.size
    n_per_row = total // (B * C)
    n_slabs = total // SLAB
    n_rows = B * C
    assert n_slabs == n_rows * ROW_SLABS
    sc_per_worker = (ROW_SLABS - TC_SLABS) // 2
    assert sc_per_worker * 2 == ROW_SLABS - TC_SLABS
    assert sc_per_worker % CH_SLABS == 0
    assert TC_SLABS % TC_CHUNK == 0
    assert n_rows * 2 == NUM_WORKERS

    pf = pred.reshape(n_slabs, 96, 96)
    tf = target.reshape(n_slabs, 96, 96)
    p4 = pred.reshape(n_rows, ROW_SLABS, 96, 96)
    t4 = target.reshape(n_rows, ROW_SLABS, 96, 96)

    mesh = plsc.VectorSubcoreMesh(core_axis_name="c", subcore_axis_name="s")
    sc_fn = pl.kernel(
        functools.partial(_sc_partials_body, sc_per_worker),
        mesh=mesh,
        out_type=jax.ShapeDtypeStruct((2, NUM_SUBCORES, 8, 128),
                                      jnp.float32),
        compiler_params=pltpu.CompilerParams(use_tc_tiling_on_sc=True),
        scratch_types=[
            pltpu.VMEM((NBUF, CH_SLABS, 96, 96), jnp.float32),
            pltpu.VMEM((NBUF, CH_SLABS, 96, 96), jnp.float32),
            pltpu.VMEM((8, 128), jnp.float32),
            pltpu.SemaphoreType.DMA,
            pltpu.SemaphoreType.DMA,
            pltpu.SemaphoreType.DMA,
            pltpu.SemaphoreType.DMA,
            pltpu.SemaphoreType.DMA,
            pltpu.SemaphoreType.DMA,
            pltpu.SemaphoreType.DMA,
            pltpu.SemaphoreType.DMA,
        ],
    )
    sc_partials = sc_fn(pf, tf)

    nch = TC_SLABS // TC_CHUNK
    tc_partials = pl.pallas_call(
        functools.partial(_tc_partials_body, nch),
        grid=(n_rows, nch),
        in_specs=[
            pl.BlockSpec((1, TC_CHUNK, 96, 96), lambda r, k: (r, k, 0, 0)),
            pl.BlockSpec((1, TC_CHUNK, 96, 96), lambda r, k: (r, k, 0, 0)),
        ],
        out_specs=pl.BlockSpec((1, 5, 96), lambda r, k: (r, 0, 0)),
        out_shape=jax.ShapeDtypeStruct((n_rows, 5, 96), jnp.float32),
        scratch_shapes=[pltpu.VMEM((5, 96, 96), jnp.float32)],
    )(p4, t4)

    out = pl.pallas_call(
        functools.partial(_combine_body, n_per_row),
        out_shape=jax.ShapeDtypeStruct((1, 1), jnp.float32),
    )(sc_partials, tc_partials)
    return out[0, 0]


# speedup vs baseline: 1.4435x; 1.0098x over previous
import functools

import jax
import jax.numpy as jnp
from jax import lax
from jax.experimental import pallas as pl
from jax.experimental.pallas import tpu as pltpu
from jax.experimental.pallas import tpu_sc as plsc

ANOMALY_THRESHOLD = 0.5
CONTRAST_EPS = 1e-08

NUM_CORES = 2
NUM_SUBCORES = 16
NUM_WORKERS = NUM_CORES * NUM_SUBCORES
LANES = 16

SLAB = 96 * 96
ROW_SLABS = 96
TC_SLABS = 56
TC_CHUNK = 56
CH_SLABS = 1
NBUF = 5
ROW_VREGS = 96 // LANES


def _sc_partials_body(sc_per_worker, pred_hbm, tgt_hbm, out_hbm,
                      pbuf, tbuf, stage, sem_p0, sem_t0, sem_p1, sem_t1,
                      sem_p2, sem_t2, sem_p3, sem_t3, sem_p4, sem_t4):
    cid = lax.axis_index("c")
    sid = lax.axis_index("s")
    wid = sid * NUM_CORES + cid
    row = wid // 2
    hlf = wid % 2
    base = row * ROW_SLABS + TC_SLABS + hlf * sc_per_worker
    nchunk = sc_per_worker // CH_SLABS

    sems = ((sem_p0, sem_t0), (sem_p1, sem_t1), (sem_p2, sem_t2),
            (sem_p3, sem_t3), (sem_p4, sem_t4))

    def start(k):
        b = k % NBUF
        sl = pl.ds(base + k * CH_SLABS, CH_SLABS)
        cp_p = pltpu.make_async_copy(pred_hbm.at[sl], pbuf.at[b], sems[b][0])
        cp_t = pltpu.make_async_copy(tgt_hbm.at[sl], tbuf.at[b], sems[b][1])
        cp_p.start()
        cp_t.start()
        return cp_p, cp_t

    zero = jnp.zeros((LANES,), jnp.float32)
    ones = jnp.ones((LANES,), jnp.float32)
    acc = (zero, zero, zero, zero, zero)

    def tree(xs):
        while len(xs) > 1:
            nxt = [xs[i] + xs[i + 1] for i in range(0, len(xs) - 1, 2)]
            if len(xs) % 2:
                nxt.append(xs[-1])
            xs = nxt
        return xs[0]

    def chunk_body(b):
        def body(r, carry):
            c_cnt, c_spa, c_sta, c_sp, c_st = carry
            ms, pms, tms, ps, ts = [], [], [], [], []
            for s in range(CH_SLABS):
                for l in range(ROW_VREGS):
                    p = pbuf[b, s, r, pl.ds(l * LANES, LANES)]
                    t = tbuf[b, s, r, pl.ds(l * LANES, LANES)]
                    anom = t > ANOMALY_THRESHOLD
                    ms.append(jnp.where(anom, ones, zero))
                    pms.append(jnp.where(anom, p, zero))
                    tms.append(jnp.where(anom, t, zero))
                    ps.append(p)
                    ts.append(t)
            return (c_cnt + tree(ms), c_spa + tree(pms), c_sta + tree(tms),
                    c_sp + tree(ps), c_st + tree(ts))
        return body

    for k in range(NBUF - 1):
        start(k)

    def group_body(g, carry):
        acc = carry
        for b in range(NBUF):
            k = g * NBUF + b
            cp_p = pltpu.make_async_copy(
                pred_hbm.at[pl.ds(base + k * CH_SLABS, CH_SLABS)],
                pbuf.at[b], sems[b][0])
            cp_t = pltpu.make_async_copy(
                tgt_hbm.at[pl.ds(base + k * CH_SLABS, CH_SLABS)],
                tbuf.at[b], sems[b][1])
            cp_p.wait()
            cp_t.wait()

            k_next = k + NBUF - 1

            @pl.when(k_next < nchunk)
            def _(k_next=k_next, b_next=(b + NBUF - 1) % NBUF):
                sl = pl.ds(base + k_next * CH_SLABS, CH_SLABS)
                pltpu.make_async_copy(pred_hbm.at[sl], pbuf.at[b_next],
                                      sems[b_next][0]).start()
                pltpu.make_async_copy(tgt_hbm.at[sl], tbuf.at[b_next],
                                      sems[b_next][1]).start()

            ch = lax.fori_loop(0, 96, chunk_body(b),
                               (zero, zero, zero, zero, zero))
            acc = tuple(a + c for a, c in zip(acc, ch))
        return acc

    assert nchunk % NBUF == 0
    acc = lax.fori_loop(0, nchunk // NBUF, group_body, acc)

    for q in range(8):
        for l in range(128 // LANES):
            stage[q, pl.ds(l * LANES, LANES)] = zero
    for q, v in enumerate(acc):
        stage[q, pl.ds(0, LANES)] = v

    pltpu.sync_copy(stage, out_hbm.at[hlf, row])


def _tc_partials_body(nch, p_ref, t_ref, o_ref, acc_ref):
    k = pl.program_id(1)
    p = p_ref[0]
    t = t_ref[0]
    m = (t > ANOMALY_THRESHOLD).astype(jnp.float32)
    parts = (m.sum(axis=0), (p * m).sum(axis=0), (t * m).sum(axis=0),
             p.sum(axis=0), t.sum(axis=0))

    for q, part in enumerate(parts):
        @pl.when(k == 0)
        def _(q=q, part=part):
            acc_ref[q] = part

        @pl.when(k > 0)
        def _(q=q, part=part):
            acc_ref[q] = acc_ref[q] + part

    @pl.when(k == nch - 1)
    def _():
        o_ref[...] = acc_ref[...].sum(axis=1).reshape(1, 5, 96)


def _combine_body(n_per_row, sc_ref, tc_ref, o_ref):
    x = sc_ref[...]
    y_sc = (x[0] + x[1])[:, :5, :LANES].sum(axis=-1)
    y_tc = tc_ref[...].sum(axis=-1)
    y = y_sc + y_tc
    cnt = y[:, 0:1]
    spa = y[:, 1:2]
    sta = y[:, 2:3]
    sp = y[:, 3:4]
    st = y[:, 4:5]
    n = jnp.float32(n_per_row)
    b_cnt = n - cnt
    safe_a = jnp.maximum(cnt, 1.0)
    safe_b = jnp.maximum(b_cnt, 1.0)
    pred_anom_mean = spa / safe_a
    pred_bg_mean = (sp - spa) / safe_b
    tgt_anom_mean = sta / safe_a
    tgt_bg_mean = (st - sta) / safe_b
    valid = (cnt > 0.0) & (b_cnt > 0.0)
    pred_contrast = pred_anom_mean - pred_bg_mean
    tgt_contrast = tgt_anom_mean - tgt_bg_mean
    ratio = pred_contrast / (tgt_contrast + CONTRAST_EPS)
    vf = valid.astype(jnp.float32)
    n_valid = jnp.sum(vf)
    mean_ratio = jnp.sum(ratio * vf) / jnp.maximum(n_valid, 1.0)
    res = jnp.where(n_valid > 0.0, mean_ratio, jnp.float32(1.0))
    o_ref[...] = jnp.full((1, 1), res, jnp.float32)


def kernel(pred, target):
    B, C = pred.shape[0], pred.shape[1]
    total = pred.size
    n_per_row = total // (B * C)
    n_slabs = total // SLAB
    n_rows = B * C
    assert n_slabs == n_rows * ROW_SLABS
    sc_per_worker = (ROW_SLABS - TC_SLABS) // 2
    assert sc_per_worker * 2 == ROW_SLABS - TC_SLABS
    assert sc_per_worker % CH_SLABS == 0
    assert TC_SLABS % TC_CHUNK == 0
    assert n_rows * 2 == NUM_WORKERS

    pf = pred.reshape(n_slabs, 96, 96)
    tf = target.reshape(n_slabs, 96, 96)
    p4 = pred.reshape(n_rows, ROW_SLABS, 96, 96)
    t4 = target.reshape(n_rows, ROW_SLABS, 96, 96)

    mesh = plsc.VectorSubcoreMesh(core_axis_name="c", subcore_axis_name="s")
    sc_fn = pl.kernel(
        functools.partial(_sc_partials_body, sc_per_worker),
        mesh=mesh,
        out_type=jax.ShapeDtypeStruct((2, NUM_SUBCORES, 8, 128),
                                      jnp.float32),
        compiler_params=pltpu.CompilerParams(use_tc_tiling_on_sc=True),
        scratch_types=[
            pltpu.VMEM((NBUF, CH_SLABS, 96, 96), jnp.float32),
            pltpu.VMEM((NBUF, CH_SLABS, 96, 96), jnp.float32),
            pltpu.VMEM((8, 128), jnp.float32),
            pltpu.SemaphoreType.DMA,
            pltpu.SemaphoreType.DMA,
            pltpu.SemaphoreType.DMA,
            pltpu.SemaphoreType.DMA,
            pltpu.SemaphoreType.DMA,
            pltpu.SemaphoreType.DMA,
            pltpu.SemaphoreType.DMA,
            pltpu.SemaphoreType.DMA,
            pltpu.SemaphoreType.DMA,
            pltpu.SemaphoreType.DMA,
        ],
    )
    sc_partials = sc_fn(pf, tf)

    nch = TC_SLABS // TC_CHUNK
    tc_partials = pl.pallas_call(
        functools.partial(_tc_partials_body, nch),
        grid=(n_rows, nch),
        in_specs=[
            pl.BlockSpec((1, TC_CHUNK, 96, 96), lambda r, k: (r, k, 0, 0)),
            pl.BlockSpec((1, TC_CHUNK, 96, 96), lambda r, k: (r, k, 0, 0)),
        ],
        out_specs=pl.BlockSpec((1, 5, 96), lambda r, k: (r, 0, 0)),
        out_shape=jax.ShapeDtypeStruct((n_rows, 5, 96), jnp.float32),
        scratch_shapes=[pltpu.VMEM((5, 96, 96), jnp.float32)],
    )(p4, t4)

    out = pl.pallas_call(
        functools.partial(_combine_body, n_per_row),
        out_shape=jax.ShapeDtypeStruct((1, 1), jnp.float32),
    )(sc_partials, tc_partials)
    return out[0, 0]


# speedup vs baseline: 1.4475x; 1.0028x over previous
import functools

import jax
import jax.numpy as jnp
from jax import lax
from jax.experimental import pallas as pl
from jax.experimental.pallas import tpu as pltpu
from jax.experimental.pallas import tpu_sc as plsc

ANOMALY_THRESHOLD = 0.5
CONTRAST_EPS = 1e-08

NUM_CORES = 2
NUM_SUBCORES = 16
NUM_WORKERS = NUM_CORES * NUM_SUBCORES
LANES = 16

SLAB = 96 * 96
ROW_SLABS = 96
TC_SLABS = 54
TC_CHUNK = 54
CH_SLABS = 1
NBUF = 5
ROW_VREGS = 96 // LANES


def _sc_partials_body(sc_per_worker, pred_hbm, tgt_hbm, out_hbm,
                      pbuf, tbuf, stage, sem_p0, sem_t0, sem_p1, sem_t1,
                      sem_p2, sem_t2, sem_p3, sem_t3, sem_p4, sem_t4):
    cid = lax.axis_index("c")
    sid = lax.axis_index("s")
    wid = sid * NUM_CORES + cid
    row = wid // 2
    hlf = wid % 2
    base = row * ROW_SLABS + TC_SLABS + hlf * sc_per_worker
    nchunk = sc_per_worker // CH_SLABS

    sems = ((sem_p0, sem_t0), (sem_p1, sem_t1), (sem_p2, sem_t2),
            (sem_p3, sem_t3), (sem_p4, sem_t4))

    def start(k):
        b = k % NBUF
        sl = pl.ds(base + k * CH_SLABS, CH_SLABS)
        cp_p = pltpu.make_async_copy(pred_hbm.at[sl], pbuf.at[b], sems[b][0])
        cp_t = pltpu.make_async_copy(tgt_hbm.at[sl], tbuf.at[b], sems[b][1])
        cp_p.start()
        cp_t.start()
        return cp_p, cp_t

    zero = jnp.zeros((LANES,), jnp.float32)
    ones = jnp.ones((LANES,), jnp.float32)
    acc = (zero, zero, zero, zero, zero)

    def tree(xs):
        while len(xs) > 1:
            nxt = [xs[i] + xs[i + 1] for i in range(0, len(xs) - 1, 2)]
            if len(xs) % 2:
                nxt.append(xs[-1])
            xs = nxt
        return xs[0]

    def chunk_body(b):
        def body(r, carry):
            c_cnt, c_spa, c_sta, c_sp, c_st = carry
            ms, pms, tms, ps, ts = [], [], [], [], []
            for s in range(CH_SLABS):
                for l in range(ROW_VREGS):
                    p = pbuf[b, s, r, pl.ds(l * LANES, LANES)]
                    t = tbuf[b, s, r, pl.ds(l * LANES, LANES)]
                    anom = t > ANOMALY_THRESHOLD
                    ms.append(jnp.where(anom, ones, zero))
                    pms.append(jnp.where(anom, p, zero))
                    tms.append(jnp.where(anom, t, zero))
                    ps.append(p)
                    ts.append(t)
            return (c_cnt + tree(ms), c_spa + tree(pms), c_sta + tree(tms),
                    c_sp + tree(ps), c_st + tree(ts))
        return body

    for k in range(NBUF - 1):
        start(k)

    def group_body(g, carry):
        acc = carry
        for b in range(NBUF):
            k = g * NBUF + b
            cp_p = pltpu.make_async_copy(
                pred_hbm.at[pl.ds(base + k * CH_SLABS, CH_SLABS)],
                pbuf.at[b], sems[b][0])
            cp_t = pltpu.make_async_copy(
                tgt_hbm.at[pl.ds(base + k * CH_SLABS, CH_SLABS)],
                tbuf.at[b], sems[b][1])
            cp_p.wait()
            cp_t.wait()

            k_next = k + NBUF - 1

            @pl.when(k_next < nchunk)
            def _(k_next=k_next, b_next=(b + NBUF - 1) % NBUF):
                sl = pl.ds(base + k_next * CH_SLABS, CH_SLABS)
                pltpu.make_async_copy(pred_hbm.at[sl], pbuf.at[b_next],
                                      sems[b_next][0]).start()
                pltpu.make_async_copy(tgt_hbm.at[sl], tbuf.at[b_next],
                                      sems[b_next][1]).start()

            ch = lax.fori_loop(0, 96, chunk_body(b),
                               (zero, zero, zero, zero, zero))
            acc = tuple(a + c for a, c in zip(acc, ch))
        return acc

    n_groups = nchunk // NBUF
    acc = lax.fori_loop(0, n_groups, group_body, acc)
    for j in range(nchunk % NBUF):
        k = n_groups * NBUF + j
        sl = pl.ds(base + k * CH_SLABS, CH_SLABS)
        pltpu.make_async_copy(pred_hbm.at[sl], pbuf.at[j], sems[j][0]).wait()
        pltpu.make_async_copy(tgt_hbm.at[sl], tbuf.at[j], sems[j][1]).wait()
        ch = lax.fori_loop(0, 96, chunk_body(j),
                           (zero, zero, zero, zero, zero))
        acc = tuple(a + c for a, c in zip(acc, ch))

    for q in range(8):
        for l in range(128 // LANES):
            stage[q, pl.ds(l * LANES, LANES)] = zero
    for q, v in enumerate(acc):
        stage[q, pl.ds(0, LANES)] = v

    pltpu.sync_copy(stage, out_hbm.at[hlf, row])


def _tc_partials_body(nch, p_ref, t_ref, o_ref, acc_ref):
    k = pl.program_id(1)
    p = p_ref[0]
    t = t_ref[0]
    m = (t > ANOMALY_THRESHOLD).astype(jnp.float32)
    parts = (m.sum(axis=0), (p * m).sum(axis=0), (t * m).sum(axis=0),
             p.sum(axis=0), t.sum(axis=0))

    for q, part in enumerate(parts):
        @pl.when(k == 0)
        def _(q=q, part=part):
            acc_ref[q] = part

        @pl.when(k > 0)
        def _(q=q, part=part):
            acc_ref[q] = acc_ref[q] + part

    @pl.when(k == nch - 1)
    def _():
        o_ref[...] = acc_ref[...].sum(axis=1).reshape(1, 5, 96)


def _combine_body(n_per_row, sc_ref, tc_ref, o_ref):
    x = sc_ref[...]
    y_sc = (x[0] + x[1])[:, :5, :LANES].sum(axis=-1)
    y_tc = tc_ref[...].sum(axis=-1)
    y = y_sc + y_tc
    cnt = y[:, 0:1]
    spa = y[:, 1:2]
    sta = y[:, 2:3]
    sp = y[:, 3:4]
    st = y[:, 4:5]
    n = jnp.float32(n_per_row)
    b_cnt = n - cnt
    safe_a = jnp.maximum(cnt, 1.0)
    safe_b = jnp.maximum(b_cnt, 1.0)
    pred_anom_mean = spa / safe_a
    pred_bg_mean = (sp - spa) / safe_b
    tgt_anom_mean = sta / safe_a
    tgt_bg_mean = (st - sta) / safe_b
    valid = (cnt > 0.0) & (b_cnt > 0.0)
    pred_contrast = pred_anom_mean - pred_bg_mean
    tgt_contrast = tgt_anom_mean - tgt_bg_mean
    ratio = pred_contrast / (tgt_contrast + CONTRAST_EPS)
    vf = valid.astype(jnp.float32)
    n_valid = jnp.sum(vf)
    mean_ratio = jnp.sum(ratio * vf) / jnp.maximum(n_valid, 1.0)
    res = jnp.where(n_valid > 0.0, mean_ratio, jnp.float32(1.0))
    o_ref[...] = jnp.full((1, 1), res, jnp.float32)


def kernel(pred, target):
    B, C = pred.shape[0], pred.shape[1]
    total = pred.size
    n_per_row = total // (B * C)
    n_slabs = total // SLAB
    n_rows = B * C
    assert n_slabs == n_rows * ROW_SLABS
    sc_per_worker = (ROW_SLABS - TC_SLABS) // 2
    assert sc_per_worker * 2 == ROW_SLABS - TC_SLABS
    assert sc_per_worker % CH_SLABS == 0
    assert TC_SLABS % TC_CHUNK == 0
    assert n_rows * 2 == NUM_WORKERS

    pf = pred.reshape(n_slabs, 96, 96)
    tf = target.reshape(n_slabs, 96, 96)
    p4 = pred.reshape(n_rows, ROW_SLABS, 96, 96)
    t4 = target.reshape(n_rows, ROW_SLABS, 96, 96)

    mesh = plsc.VectorSubcoreMesh(core_axis_name="c", subcore_axis_name="s")
    sc_fn = pl.kernel(
        functools.partial(_sc_partials_body, sc_per_worker),
        mesh=mesh,
        out_type=jax.ShapeDtypeStruct((2, NUM_SUBCORES, 8, 128),
                                      jnp.float32),
        compiler_params=pltpu.CompilerParams(use_tc_tiling_on_sc=True),
        scratch_types=[
            pltpu.VMEM((NBUF, CH_SLABS, 96, 96), jnp.float32),
            pltpu.VMEM((NBUF, CH_SLABS, 96, 96), jnp.float32),
            pltpu.VMEM((8, 128), jnp.float32),
            pltpu.SemaphoreType.DMA,
            pltpu.SemaphoreType.DMA,
            pltpu.SemaphoreType.DMA,
            pltpu.SemaphoreType.DMA,
            pltpu.SemaphoreType.DMA,
            pltpu.SemaphoreType.DMA,
            pltpu.SemaphoreType.DMA,
            pltpu.SemaphoreType.DMA,
            pltpu.SemaphoreType.DMA,
            pltpu.SemaphoreType.DMA,
        ],
    )
    sc_partials = sc_fn(pf, tf)

    nch = TC_SLABS // TC_CHUNK
    tc_partials = pl.pallas_call(
        functools.partial(_tc_partials_body, nch),
        grid=(n_rows, nch),
        in_specs=[
            pl.BlockSpec((1, TC_CHUNK, 96, 96), lambda r, k: (r, k, 0, 0)),
            pl.BlockSpec((1, TC_CHUNK, 96, 96), lambda r, k: (r, k, 0, 0)),
        ],
        out_specs=pl.BlockSpec((1, 5, 96), lambda r, k: (r, 0, 0)),
        out_shape=jax.ShapeDtypeStruct((n_rows, 5, 96), jnp.float32),
        scratch_shapes=[pltpu.VMEM((5, 96, 96), jnp.float32)],
    )(p4, t4)

    out = pl.pallas_call(
        functools.partial(_combine_body, n_per_row),
        out_shape=jax.ShapeDtypeStruct((1, 1), jnp.float32),
    )(sc_partials, tc_partials)
    return out[0, 0]
